# baseline (speedup 1.0000x reference)
import jax
import jax.numpy as jnp
from jax import lax
from jax.experimental import pallas as pl
from jax.experimental.pallas import tpu as pltpu
from jax.experimental.pallas import tpu_sc as plsc

N = 320000
FEAT_DIM = 128
HIDDEN = 512
LATENT = 128
NUM_SEGMENTS = 10000
NJ = LATENT // 16

NC = 2
NS = 16
NW = NC * NS
CHUNK = 400
NBLK = CHUNK // 16
RING = 8

HALF_A = 153600
HALF_B = 166400

ROW_TILE = 800
SEG_TILE = 1000


def _mlp_body(x_ref, w1_ref, b1_ref, w2_ref, b2_ref, o_ref):
    h = jnp.maximum(
        jnp.dot(x_ref[...].astype(jnp.bfloat16), w1_ref[...],
                preferred_element_type=jnp.float32)
        + b1_ref[...], 0.0)
    e = jnp.maximum(
        jnp.dot(h.astype(jnp.bfloat16), w2_ref[...],
                preferred_element_type=jnp.float32)
        + b2_ref[...], 0.0)
    o_ref[...] = e


def _mlp(feat, W1, b1, W2, b2, n_rows):
    grid = (n_rows // ROW_TILE,)
    return pl.pallas_call(
        _mlp_body,
        grid=grid,
        in_specs=[
            pl.BlockSpec((ROW_TILE, FEAT_DIM), lambda i: (i, 0)),
            pl.BlockSpec((FEAT_DIM, HIDDEN), lambda i: (0, 0)),
            pl.BlockSpec((1, HIDDEN), lambda i: (0, 0)),
            pl.BlockSpec((HIDDEN, LATENT), lambda i: (0, 0)),
            pl.BlockSpec((1, LATENT), lambda i: (0, 0)),
        ],
        out_specs=pl.BlockSpec((ROW_TILE, LATENT), lambda i: (i, 0)),
        out_shape=jax.ShapeDtypeStruct((n_rows, LATENT), jnp.float32),
    )(feat, W1, b1, W2, b2)


def _segmax_body(rows_per_w, nchunk,
                 emb_hbm, idx_hbm, direct_hbm, partials_hbm, pids_hbm,
                 idx_v, buf_v, mbuf_v, stage_v, zrow_v, pid_v,
                 flush_sem, chunk_sem):
    w = lax.axis_index("s") * NC + lax.axis_index("c")
    base = w * rows_per_w
    zero16i = jnp.zeros((16,), jnp.int32)
    zero16f = jnp.zeros((16,), jnp.float32)

    idx_v[pl.ds(0, 16)] = zero16i - 1
    pltpu.sync_copy(idx_hbm.at[pl.ds(base, rows_per_w)],
                    idx_v.at[pl.ds(16, rows_per_w)])

    def idx_at(r):
        return idx_v[pl.ds(16 + r, 16)][0]

    for j in range(NJ):
        zrow_v[pl.ds(j * 16, 16)] = zero16f
        mbuf_v[pl.ds(j * 16, 16)] = zero16f

    def chunk_src(c):
        return emb_hbm.at[pl.ds((base + c * CHUNK) * LATENT, CHUNK * LATENT)]

    def chunk_dst(c):
        return buf_v.at[pl.ds(lax.rem(c, 2) * (CHUNK * LATENT),
                              CHUNK * LATENT)]

    def start_chunk(c):
        pltpu.async_copy(chunk_src(c), chunk_dst(c),
                         chunk_sem.at[lax.rem(c, 2)])

    def wait_chunk(c):
        pltpu.make_async_copy(chunk_src(c), chunk_dst(c),
                              chunk_sem.at[lax.rem(c, 2)]).wait()

    def stage_slot(slot):
        return stage_v.at[pl.ds(slot * LATENT, LATENT)]

    def flush(cur, m, first_open, k):
        slot = lax.rem(k, RING)

        def wait_slot():
            pltpu.make_async_copy(stage_slot(slot),
                                  direct_hbm.at[pl.ds(0, LATENT)],
                                  flush_sem.at[slot]).wait()

        lax.cond(jnp.logical_and(first_open == 0, k >= RING),
                 wait_slot, lambda: None)
        for j in range(NJ):
            stage_v[pl.ds(slot * LATENT + j * 16, 16)] = m[j]

        def to_partial():
            pltpu.sync_copy(stage_slot(slot),
                            partials_hbm.at[pl.ds(2 * w * LATENT, LATENT)])

        def to_direct():
            pltpu.async_copy(stage_slot(slot),
                             direct_hbm.at[pl.ds(cur * LATENT, LATENT)],
                             flush_sem.at[slot])

        lax.cond(first_open == 1, to_partial, to_direct)
        return jnp.where(first_open == 1, k, k + 1)

    def zero_gap(lo, hi):
        def body(g, _):
            pltpu.sync_copy(zrow_v, direct_hbm.at[pl.ds(g * LATENT, LATENT)])
            return 0
        lax.fori_loop(lo, hi, body, 0)

    def load_m():
        return tuple(mbuf_v[pl.ds(j * 16, 16)] for j in range(NJ))

    def store_m(m):
        for j in range(NJ):
            mbuf_v[pl.ds(j * 16, 16)] = m[j]

    def row_vals(boff, r):
        return tuple(buf_v[pl.ds(boff + r * LATENT + j * 16, 16)]
                     for j in range(NJ))

    def block_body(b, carry):
        c, cur, first_open, k = carry
        g = b * 16
        boff = (lax.rem(c, 2) * CHUNK + (g - c * CHUNK)) * LATENT
        a_vec = idx_v[pl.ds(16 + g, 16)]
        p_vec = idx_v[pl.ds(15 + g, 16)]
        nb = plsc.all_reduce_population_count(a_vec != p_vec)[0]

        def fast():
            m = load_m()
            rows = [row_vals(boff, r) for r in range(16)]
            while len(rows) > 1:
                rows = [tuple(jnp.maximum(x[j], y[j]) for j in range(NJ))
                        for x, y in zip(rows[::2], rows[1::2])]
            store_m(tuple(jnp.maximum(m[j], rows[0][j]) for j in range(NJ)))
            return cur, first_open, k

        def slow():
            def row_body(r, rcarry):
                rcur, ropen, rk = rcarry
                s = idx_at(g + r)
                v = row_vals(boff, r)
                changed = s != rcur
                m = load_m()

                def on_change(_):
                    nk = flush(rcur, m, ropen, rk)
                    zero_gap(rcur + 1, s)
                    return nk

                nk = lax.cond(changed, on_change, lambda _: rk, 0)
                store_m(tuple(
                    jnp.where(changed, v[j], jnp.maximum(m[j], v[j]))
                    for j in range(NJ)))
                return (jnp.where(changed, s, rcur),
                        jnp.where(changed, jnp.int32(0), ropen),
                        nk)

            return lax.fori_loop(0, 16, row_body, (cur, first_open, k))

        cur2, open2, k2 = lax.cond(nb == 0, fast, slow)
        return c, cur2, open2, k2

    def chunk_body(c, carry):
        def prefetch():
            start_chunk(c + 1)
        lax.cond(c < nchunk - 1, prefetch, lambda: None)
        wait_chunk(c)
        cur, first_open, k = carry
        _, cur, first_open, k = lax.fori_loop(
            c * NBLK, (c + 1) * NBLK, block_body, (c, cur, first_open, k))
        return cur, first_open, k

    start_chunk(0)
    init = (idx_at(0), jnp.int32(1), jnp.int32(0))
    cur, first_open, k = lax.fori_loop(0, nchunk, chunk_body, init)

    for s in range(RING):
        def drain():
            pltpu.make_async_copy(stage_slot(s),
                                  direct_hbm.at[pl.ds(0, LATENT)],
                                  flush_sem.at[s]).wait()
        lax.cond(k > s, drain, lambda: None)

    m = load_m()
    for j in range(NJ):
        stage_v[pl.ds(j * 16, 16)] = m[j]
    pltpu.sync_copy(stage_v.at[pl.ds(0, LATENT)],
                    partials_hbm.at[pl.ds((2 * w + 1) * LATENT, LATENT)])

    def also_first():
        pltpu.sync_copy(stage_v.at[pl.ds(0, LATENT)],
                        partials_hbm.at[pl.ds(2 * w * LATENT, LATENT)])

    lax.cond(first_open == 1, also_first, lambda: None)

    lane = lax.broadcasted_iota(jnp.int32, (16,), 0)
    pid_v[...] = jnp.where(lane == 0, idx_at(0),
                           jnp.where(lane == 1, cur, 0))
    pltpu.sync_copy(pid_v, pids_hbm.at[pl.ds(w * 16, 16)])


def _segmax(emb, idx, n_rows):
    import functools
    rows_per_w = n_rows // NW
    nchunk = rows_per_w // CHUNK
    mesh = plsc.VectorSubcoreMesh(core_axis_name="c", subcore_axis_name="s")
    f = pl.kernel(
        functools.partial(_segmax_body, rows_per_w, nchunk),
        out_type=(
            jax.ShapeDtypeStruct((NUM_SEGMENTS * LATENT,), jnp.float32),
            jax.ShapeDtypeStruct((2 * NW * LATENT,), jnp.float32),
            jax.ShapeDtypeStruct((NW * 16,), jnp.int32),
        ),
        mesh=mesh,
        compiler_params=pltpu.CompilerParams(use_tc_tiling_on_sc=False,
                                             needs_layout_passes=False),
        scratch_types=[
            pltpu.VMEM((32 + rows_per_w + 16,), jnp.int32),
            pltpu.VMEM((2 * CHUNK * LATENT,), jnp.float32),
            pltpu.VMEM((LATENT,), jnp.float32),
            pltpu.VMEM((RING * LATENT,), jnp.float32),
            pltpu.VMEM((LATENT,), jnp.float32),
            pltpu.VMEM((16,), jnp.int32),
            pltpu.SemaphoreType.DMA((RING,)),
            pltpu.SemaphoreType.DMA((2,)),
        ],
    )
    return f(emb.reshape(-1), idx)


PNUM = 4 * NW


def _pid_at(pid_ref, k):
    return pid_ref[16 * (k // 2) + (k % 2)]


def _final_body(d1_ref, d2_ref, p_ref, pid_ref, w3_ref, b3_ref, o_ref,
                pm_ref, val_ref):
    i = pl.program_id(0)
    sid = lax.broadcasted_iota(jnp.int32, (SEG_TILE, 1), 0) + i * SEG_TILE

    @pl.when(i == 0)
    def _():
        krow = lax.broadcasted_iota(jnp.int32, (PNUM, 1), 0)
        pids_col = jnp.zeros((PNUM, 1), jnp.int32)
        for k in range(PNUM):
            pids_col = jnp.where(krow == k, _pid_at(pid_ref, k), pids_col)
        p = p_ref[...]
        pm = p
        for k in range(PNUM):
            m = jnp.max(jnp.where(pids_col == _pid_at(pid_ref, k), p, 0.0),
                        axis=0, keepdims=True)
            pm = jnp.where(krow == k, m, pm)
        pm_ref[...] = pm

    def half_val(d_ref, pid_base):
        clear = jnp.zeros((SEG_TILE, 1), jnp.bool_)
        for w in range(NW + 1):
            lo = (jnp.int32(-1) if w == 0
                  else pid_ref[pid_base + 16 * (w - 1) + 1])
            hi = (jnp.int32(NUM_SEGMENTS) if w == NW
                  else pid_ref[pid_base + 16 * w])
            clear = jnp.logical_or(clear,
                                   jnp.logical_and(sid > lo, sid < hi))
        return jnp.where(clear, 0.0, d_ref[...])

    val_ref[...] = jnp.maximum(half_val(d1_ref, 0),
                               half_val(d2_ref, 16 * NW))

    for k in range(PNUM):
        pid = _pid_at(pid_ref, k)

        @pl.when(jnp.logical_and(pid >= i * SEG_TILE,
                                 pid < (i + 1) * SEG_TILE))
        def _():
            val_ref[pl.ds(pid - i * SEG_TILE, 1), :] = pm_ref[k:k + 1, :]

    o_ref[...] = (jnp.dot(val_ref[...].astype(jnp.bfloat16), w3_ref[...],
                          preferred_element_type=jnp.float32)
                  + b3_ref[...])


def _final(d1, d2, partials, pids, W3, b3):
    grid = (NUM_SEGMENTS // SEG_TILE,)
    return pl.pallas_call(
        _final_body,
        grid=grid,
        in_specs=[
            pl.BlockSpec((SEG_TILE, LATENT), lambda i: (i, 0)),
            pl.BlockSpec((SEG_TILE, LATENT), lambda i: (i, 0)),
            pl.BlockSpec((PNUM, LATENT), lambda i: (0, 0)),
            pl.BlockSpec(memory_space=pltpu.SMEM),
            pl.BlockSpec((LATENT, LATENT), lambda i: (0, 0)),
            pl.BlockSpec((1, LATENT), lambda i: (0, 0)),
        ],
        out_specs=pl.BlockSpec((SEG_TILE, LATENT), lambda i: (i, 0)),
        out_shape=jax.ShapeDtypeStruct((NUM_SEGMENTS, LATENT), jnp.float32),
        scratch_shapes=[
            pltpu.VMEM((PNUM, LATENT), jnp.float32),
            pltpu.VMEM((SEG_TILE, LATENT), jnp.float32),
        ],
    )(d1, d2, partials, pids, W3, b3)


def kernel(feat, traj_inbatch_index, W1, b1, W2, b2, W3, b3):
    idx = traj_inbatch_index.astype(jnp.int32)
    W1b = W1.astype(jnp.bfloat16)
    W2b = W2.astype(jnp.bfloat16)
    b1r = b1.reshape(1, HIDDEN)
    b2r = b2.reshape(1, LATENT)

    emb_a = _mlp(feat[:HALF_A], W1b, b1r, W2b, b2r, HALF_A)
    d1, p1, i1 = _segmax(emb_a, idx[:HALF_A], HALF_A)
    emb_b = _mlp(feat[HALF_A:], W1b, b1r, W2b, b2r, HALF_B)
    d2, p2, i2 = _segmax(emb_b, idx[HALF_A:], HALF_B)

    partials = jnp.concatenate([p1, p2]).reshape(PNUM, LATENT)
    pids = jnp.concatenate([i1, i2])
    return _final(d1.reshape(NUM_SEGMENTS, LATENT),
                  d2.reshape(NUM_SEGMENTS, LATENT),
                  partials, pids, W3.astype(jnp.bfloat16),
                  b3.reshape(1, LATENT))


# speedup vs baseline: 1.1830x; 1.1830x over previous
import jax
import jax.numpy as jnp
from jax import lax
from jax.experimental import pallas as pl
from jax.experimental.pallas import tpu as pltpu
from jax.experimental.pallas import tpu_sc as plsc

N = 320000
FEAT_DIM = 128
HIDDEN = 512
LATENT = 128
NUM_SEGMENTS = 10000
NJ = LATENT // 16

NC = 2
NS = 16
NW = NC * NS
CHUNK = 400
NBLK = CHUNK // 16
RING = 8

HALF_A = 153600
HALF_B = 166400

ROW_TILE = 800
SEG_TILE = 1000


def _mlp_body(x_ref, w1_ref, b1_ref, w2_ref, b2_ref, o_ref):
    h = jnp.maximum(
        jnp.dot(x_ref[...].astype(jnp.bfloat16), w1_ref[...],
                preferred_element_type=jnp.float32)
        + b1_ref[...], 0.0)
    e = jnp.maximum(
        jnp.dot(h.astype(jnp.bfloat16), w2_ref[...],
                preferred_element_type=jnp.float32)
        + b2_ref[...], 0.0)
    o_ref[...] = e


def _mlp(feat, W1, b1, W2, b2, n_rows, row0):
    grid = (n_rows // ROW_TILE,)
    t0 = row0 // ROW_TILE
    return pl.pallas_call(
        _mlp_body,
        grid=grid,
        in_specs=[
            pl.BlockSpec((ROW_TILE, FEAT_DIM), lambda i: (i + t0, 0)),
            pl.BlockSpec((FEAT_DIM, HIDDEN), lambda i: (0, 0)),
            pl.BlockSpec((1, HIDDEN), lambda i: (0, 0)),
            pl.BlockSpec((HIDDEN, LATENT), lambda i: (0, 0)),
            pl.BlockSpec((1, LATENT), lambda i: (0, 0)),
        ],
        out_specs=pl.BlockSpec((ROW_TILE, LATENT), lambda i: (i, 0)),
        out_shape=jax.ShapeDtypeStruct((n_rows, LATENT), jnp.float32),
    )(feat, W1, b1, W2, b2)


def _segmax_body(rows_per_w, nchunk, row0,
                 emb_hbm, idx_hbm, direct_hbm, partials_hbm, pids_hbm,
                 idx_v, buf_v, mbuf_v, stage_v, zrow_v, pid_v,
                 flush_sem, chunk_sem):
    w = lax.axis_index("s") * NC + lax.axis_index("c")
    base = w * rows_per_w
    zero16i = jnp.zeros((16,), jnp.int32)
    zero16f = jnp.zeros((16,), jnp.float32)

    idx_v[pl.ds(0, 16)] = zero16i - 1
    pltpu.sync_copy(idx_hbm.at[pl.ds(row0 + base, rows_per_w)],
                    idx_v.at[pl.ds(16, rows_per_w)])

    def idx_at(r):
        return idx_v[pl.ds(16 + r, 16)][0]

    for j in range(NJ):
        zrow_v[pl.ds(j * 16, 16)] = zero16f
        mbuf_v[pl.ds(j * 16, 16)] = zero16f

    def chunk_src(c):
        return emb_hbm.at[pl.ds((base + c * CHUNK) * LATENT, CHUNK * LATENT)]

    def chunk_dst(c):
        return buf_v.at[pl.ds(lax.rem(c, 2) * (CHUNK * LATENT),
                              CHUNK * LATENT)]

    def start_chunk(c):
        pltpu.async_copy(chunk_src(c), chunk_dst(c),
                         chunk_sem.at[lax.rem(c, 2)])

    def wait_chunk(c):
        pltpu.make_async_copy(chunk_src(c), chunk_dst(c),
                              chunk_sem.at[lax.rem(c, 2)]).wait()

    def stage_slot(slot):
        return stage_v.at[pl.ds(slot * LATENT, LATENT)]

    def flush(cur, m, first_open, k):
        slot = lax.rem(k, RING)

        def wait_slot():
            pltpu.make_async_copy(stage_slot(slot),
                                  direct_hbm.at[pl.ds(0, LATENT)],
                                  flush_sem.at[slot]).wait()

        lax.cond(jnp.logical_and(first_open == 0, k >= RING),
                 wait_slot, lambda: None)
        for j in range(NJ):
            stage_v[pl.ds(slot * LATENT + j * 16, 16)] = m[j]

        def to_partial():
            pltpu.sync_copy(stage_slot(slot),
                            partials_hbm.at[pl.ds(2 * w * LATENT, LATENT)])

        def to_direct():
            pltpu.async_copy(stage_slot(slot),
                             direct_hbm.at[pl.ds(cur * LATENT, LATENT)],
                             flush_sem.at[slot])

        lax.cond(first_open == 1, to_partial, to_direct)
        return jnp.where(first_open == 1, k, k + 1)

    def zero_gap(lo, hi):
        def body(g, _):
            pltpu.sync_copy(zrow_v, direct_hbm.at[pl.ds(g * LATENT, LATENT)])
            return 0
        lax.fori_loop(lo, hi, body, 0)

    def load_m():
        return tuple(mbuf_v[pl.ds(j * 16, 16)] for j in range(NJ))

    def store_m(m):
        for j in range(NJ):
            mbuf_v[pl.ds(j * 16, 16)] = m[j]

    def row_vals(boff, r):
        return tuple(buf_v[pl.ds(boff + r * LATENT + j * 16, 16)]
                     for j in range(NJ))

    def block_body(b, carry):
        c, cur, first_open, k = carry
        g = b * 16
        boff = (lax.rem(c, 2) * CHUNK + (g - c * CHUNK)) * LATENT
        a_vec = idx_v[pl.ds(16 + g, 16)]
        p_vec = idx_v[pl.ds(15 + g, 16)]
        nb = plsc.all_reduce_population_count(a_vec != p_vec)[0]

        def fast():
            m = load_m()
            rows = [row_vals(boff, r) for r in range(16)]
            while len(rows) > 1:
                rows = [tuple(jnp.maximum(x[j], y[j]) for j in range(NJ))
                        for x, y in zip(rows[::2], rows[1::2])]
            store_m(tuple(jnp.maximum(m[j], rows[0][j]) for j in range(NJ)))
            return cur, first_open, k

        def slow():
            def row_body(r, rcarry):
                rcur, ropen, rk = rcarry
                s = idx_at(g + r)
                v = row_vals(boff, r)
                changed = s != rcur
                m = load_m()

                def on_change(_):
                    nk = flush(rcur, m, ropen, rk)
                    zero_gap(rcur + 1, s)
                    return nk

                nk = lax.cond(changed, on_change, lambda _: rk, 0)
                store_m(tuple(
                    jnp.where(changed, v[j], jnp.maximum(m[j], v[j]))
                    for j in range(NJ)))
                return (jnp.where(changed, s, rcur),
                        jnp.where(changed, jnp.int32(0), ropen),
                        nk)

            return lax.fori_loop(0, 16, row_body, (cur, first_open, k))

        cur2, open2, k2 = lax.cond(nb == 0, fast, slow)
        return c, cur2, open2, k2

    def chunk_body(c, carry):
        def prefetch():
            start_chunk(c + 1)
        lax.cond(c < nchunk - 1, prefetch, lambda: None)
        wait_chunk(c)
        cur, first_open, k = carry
        _, cur, first_open, k = lax.fori_loop(
            c * NBLK, (c + 1) * NBLK, block_body, (c, cur, first_open, k))
        return cur, first_open, k

    start_chunk(0)
    init = (idx_at(0), jnp.int32(1), jnp.int32(0))
    cur, first_open, k = lax.fori_loop(0, nchunk, chunk_body, init)

    for s in range(RING):
        def drain():
            pltpu.make_async_copy(stage_slot(s),
                                  direct_hbm.at[pl.ds(0, LATENT)],
                                  flush_sem.at[s]).wait()
        lax.cond(k > s, drain, lambda: None)

    m = load_m()
    for j in range(NJ):
        stage_v[pl.ds(j * 16, 16)] = m[j]
    pltpu.sync_copy(stage_v.at[pl.ds(0, LATENT)],
                    partials_hbm.at[pl.ds((2 * w + 1) * LATENT, LATENT)])

    def also_first():
        pltpu.sync_copy(stage_v.at[pl.ds(0, LATENT)],
                        partials_hbm.at[pl.ds(2 * w * LATENT, LATENT)])

    lax.cond(first_open == 1, also_first, lambda: None)

    lane = lax.broadcasted_iota(jnp.int32, (16,), 0)
    pid_v[...] = jnp.where(lane == 0, idx_at(0),
                           jnp.where(lane == 1, cur, 0))
    pltpu.sync_copy(pid_v, pids_hbm.at[pl.ds(w * 16, 16)])


def _segmax(emb, idx, n_rows, row0):
    import functools
    rows_per_w = n_rows // NW
    nchunk = rows_per_w // CHUNK
    mesh = plsc.VectorSubcoreMesh(core_axis_name="c", subcore_axis_name="s")
    f = pl.kernel(
        functools.partial(_segmax_body, rows_per_w, nchunk, row0),
        out_type=(
            jax.ShapeDtypeStruct((NUM_SEGMENTS * LATENT,), jnp.float32),
            jax.ShapeDtypeStruct((2 * NW * LATENT,), jnp.float32),
            jax.ShapeDtypeStruct((NW * 16,), jnp.int32),
        ),
        mesh=mesh,
        compiler_params=pltpu.CompilerParams(use_tc_tiling_on_sc=False,
                                             needs_layout_passes=False),
        scratch_types=[
            pltpu.VMEM((32 + rows_per_w + 16,), jnp.int32),
            pltpu.VMEM((2 * CHUNK * LATENT,), jnp.float32),
            pltpu.VMEM((LATENT,), jnp.float32),
            pltpu.VMEM((RING * LATENT,), jnp.float32),
            pltpu.VMEM((LATENT,), jnp.float32),
            pltpu.VMEM((16,), jnp.int32),
            pltpu.SemaphoreType.DMA((RING,)),
            pltpu.SemaphoreType.DMA((2,)),
        ],
    )
    return f(emb.reshape(-1), idx)


PNUM = 4 * NW


def _pid_at(pid_ref, k):
    return pid_ref[16 * (k // 2) + (k % 2)]


def _final_body(d1_ref, d2_ref, p_ref, pid_ref, w3_ref, b3_ref, o_ref,
                pm_ref, val_ref):
    i = pl.program_id(0)
    sid = lax.broadcasted_iota(jnp.int32, (SEG_TILE, 1), 0) + i * SEG_TILE

    @pl.when(i == 0)
    def _():
        krow = lax.broadcasted_iota(jnp.int32, (PNUM, 1), 0)
        pids_col = jnp.zeros((PNUM, 1), jnp.int32)
        for k in range(PNUM):
            pids_col = jnp.where(krow == k, _pid_at(pid_ref, k), pids_col)
        p = p_ref[...]
        pm = p
        for k in range(PNUM):
            m = jnp.max(jnp.where(pids_col == _pid_at(pid_ref, k), p, 0.0),
                        axis=0, keepdims=True)
            pm = jnp.where(krow == k, m, pm)
        pm_ref[...] = pm

    def half_val(d_ref, pid_base):
        clear = jnp.zeros((SEG_TILE, 1), jnp.bool_)
        for w in range(NW + 1):
            lo = (jnp.int32(-1) if w == 0
                  else pid_ref[pid_base + 16 * (w - 1) + 1])
            hi = (jnp.int32(NUM_SEGMENTS) if w == NW
                  else pid_ref[pid_base + 16 * w])
            clear = jnp.logical_or(clear,
                                   jnp.logical_and(sid > lo, sid < hi))
        return jnp.where(clear, 0.0, d_ref[...])

    val_ref[...] = jnp.maximum(half_val(d1_ref, 0),
                               half_val(d2_ref, 16 * NW))

    for k in range(PNUM):
        pid = _pid_at(pid_ref, k)

        @pl.when(jnp.logical_and(pid >= i * SEG_TILE,
                                 pid < (i + 1) * SEG_TILE))
        def _():
            val_ref[pl.ds(pid - i * SEG_TILE, 1), :] = pm_ref[k:k + 1, :]

    o_ref[...] = (jnp.dot(val_ref[...].astype(jnp.bfloat16), w3_ref[...],
                          preferred_element_type=jnp.float32)
                  + b3_ref[...])


def _final(d1, d2, partials, pids, W3, b3):
    grid = (NUM_SEGMENTS // SEG_TILE,)
    return pl.pallas_call(
        _final_body,
        grid=grid,
        in_specs=[
            pl.BlockSpec((SEG_TILE, LATENT), lambda i: (i, 0)),
            pl.BlockSpec((SEG_TILE, LATENT), lambda i: (i, 0)),
            pl.BlockSpec((PNUM, LATENT), lambda i: (0, 0)),
            pl.BlockSpec(memory_space=pltpu.SMEM),
            pl.BlockSpec((LATENT, LATENT), lambda i: (0, 0)),
            pl.BlockSpec((1, LATENT), lambda i: (0, 0)),
        ],
        out_specs=pl.BlockSpec((SEG_TILE, LATENT), lambda i: (i, 0)),
        out_shape=jax.ShapeDtypeStruct((NUM_SEGMENTS, LATENT), jnp.float32),
        scratch_shapes=[
            pltpu.VMEM((PNUM, LATENT), jnp.float32),
            pltpu.VMEM((SEG_TILE, LATENT), jnp.float32),
        ],
    )(d1, d2, partials, pids, W3, b3)


def kernel(feat, traj_inbatch_index, W1, b1, W2, b2, W3, b3):
    idx = traj_inbatch_index.astype(jnp.int32)
    W1b = W1.astype(jnp.bfloat16)
    W2b = W2.astype(jnp.bfloat16)
    b1r = b1.reshape(1, HIDDEN)
    b2r = b2.reshape(1, LATENT)

    emb_a = _mlp(feat, W1b, b1r, W2b, b2r, HALF_A, 0)
    d1, p1, i1 = _segmax(emb_a, idx, HALF_A, 0)
    emb_b = _mlp(feat, W1b, b1r, W2b, b2r, HALF_B, HALF_A)
    d2, p2, i2 = _segmax(emb_b, idx, HALF_B, HALF_A)

    partials = jnp.concatenate([p1, p2]).reshape(PNUM, LATENT)
    pids = jnp.concatenate([i1, i2])
    return _final(d1.reshape(NUM_SEGMENTS, LATENT),
                  d2.reshape(NUM_SEGMENTS, LATENT),
                  partials, pids, W3.astype(jnp.bfloat16),
                  b3.reshape(1, LATENT))
